# baseline (speedup 1.0000x reference)
import functools

import jax
import jax.numpy as jnp
from jax import lax
from jax.experimental import pallas as pl
from jax.experimental.pallas import tpu as pltpu
from jax.experimental.pallas import tpu_sc as plsc

VOCAB = 100000
EMBED_DIM = 64
CONTEXT = 200
HIDDEN = 128

_SC_NC = 2
_SC_NS = 16
_ROWS_PER_WORKER = 8
_ACTIVE_WORKERS = CONTEXT // _ROWS_PER_WORKER


def _sc_gather(embT, idx):
    mesh = plsc.VectorSubcoreMesh(core_axis_name="c", subcore_axis_name="s")

    @functools.partial(
        pl.kernel,
        mesh=mesh,
        out_type=jax.ShapeDtypeStruct((CONTEXT * EMBED_DIM,), jnp.float32),
        scratch_types=[
            pltpu.VMEM((16,), jnp.int32),
            pltpu.VMEM((_ROWS_PER_WORKER, EMBED_DIM, 128), jnp.float32),
            pltpu.VMEM((_ROWS_PER_WORKER * EMBED_DIM,), jnp.float32),
            pltpu.SemaphoreType.DMA,
        ],
        compiler_params=pltpu.CompilerParams(needs_layout_passes=False),
    )
    def k(embT_hbm, idx_hbm, out_hbm, idx_v, slab_v, rows_v, sem):
        wid = lax.axis_index("s") * _SC_NC + lax.axis_index("c")
        base = wid * _ROWS_PER_WORKER

        @pl.when(wid < _ACTIVE_WORKERS)
        def _():
            pltpu.sync_copy(idx_hbm.at[pl.ds(base, _ROWS_PER_WORKER)],
                            idx_v.at[pl.ds(0, _ROWS_PER_WORKER)])
            idx_vec = idx_v[...]
            copies = []
            for j in range(_ROWS_PER_WORKER):
                c0 = (idx_vec[j] // 128) * 128
                copies.append(pltpu.async_copy(
                    embT_hbm.at[:, pl.ds(c0, 128)], slab_v.at[j], sem))
            for c in copies:
                c.wait()
            for j in range(_ROWS_PER_WORKER):
                lane = idx_vec[j] % 128
                lane_vec = jnp.full((16,), lane, jnp.int32)
                j_vec = jnp.full((16,), j, jnp.int32)
                for ch in range(EMBED_DIM // 16):
                    d_vec = lax.iota(jnp.int32, 16) + 16 * ch
                    vals = plsc.load_gather(slab_v, [j_vec, d_vec, lane_vec])
                    rows_v[pl.ds(j * EMBED_DIM + 16 * ch, 16)] = vals
            pltpu.sync_copy(
                rows_v, out_hbm.at[pl.ds(base * EMBED_DIM,
                                         _ROWS_PER_WORKER * EMBED_DIM)])

    return k(embT, idx)


_BV = 4096
_NB = -(-VOCAB // _BV)


def _mlp_body(e_ref, W1_ref, b1_ref, W2_ref, b2_ref, out_ref,
              acc_ref, h_ref, m_ref, s_ref):
    i = pl.program_id(0)

    @pl.when(i == 0)
    def _():
        h = lax.dot_general(
            e_ref[...], W1_ref[...],
            dimension_numbers=(((1,), (1,)), ((), ())),
            preferred_element_type=jnp.float32,
        ) + b1_ref[...]
        h_ref[...] = jnp.reshape(jnp.maximum(h, 0.0), (HIDDEN, 1)
                                 ).astype(jnp.bfloat16)
        m_ref[...] = jnp.full((1, 1), -jnp.inf, jnp.float32)
        s_ref[...] = jnp.zeros((1, 1), jnp.float32)

    logits_col = lax.dot_general(
        W2_ref[...].astype(jnp.bfloat16), h_ref[...],
        dimension_numbers=(((1,), (0,)), ((), ())),
        preferred_element_type=jnp.float32,
    )
    logits = jnp.transpose(logits_col, (1, 0)) + b2_ref[...]
    acc_ref[:, pl.ds(i * _BV, _BV)] = logits

    col = i * _BV + lax.broadcasted_iota(jnp.int32, (1, _BV), 1)
    masked = jnp.where(col < VOCAB, logits, -jnp.inf)

    m_old = m_ref[...]
    m_new = jnp.maximum(m_old, jnp.max(masked, axis=(0, 1), keepdims=True))
    s_ref[...] = (s_ref[...] * jnp.exp(m_old - m_new)
                  + jnp.sum(jnp.exp(masked - m_new), axis=(0, 1), keepdims=True))
    m_ref[...] = m_new

    @pl.when(i == _NB - 1)
    def _():
        lse = m_ref[...] + jnp.log(s_ref[...])
        out_ref[...] = acc_ref[:, :VOCAB] - lse


def _mlp_logits(embeds, W1, b1, W2, b2):
    return pl.pallas_call(
        _mlp_body,
        grid=(_NB,),
        in_specs=[
            pl.BlockSpec((1, CONTEXT * EMBED_DIM), lambda i: (0, 0)),
            pl.BlockSpec((HIDDEN, CONTEXT * EMBED_DIM), lambda i: (0, 0)),
            pl.BlockSpec((1, HIDDEN), lambda i: (0, 0)),
            pl.BlockSpec((_BV, HIDDEN), lambda i: (i, 0)),
            pl.BlockSpec((1, _BV), lambda i: (0, i)),
        ],
        out_specs=pl.BlockSpec((1, VOCAB), lambda i: (0, 0)),
        out_shape=jax.ShapeDtypeStruct((1, VOCAB), jnp.float32),
        scratch_shapes=[
            pltpu.VMEM((1, _BV * _NB), jnp.float32),
            pltpu.VMEM((HIDDEN, 1), jnp.bfloat16),
            pltpu.VMEM((1, 1), jnp.float32),
            pltpu.VMEM((1, 1), jnp.float32),
        ],
    )(embeds, W1, b1, W2, b2)


def kernel(inputs, emb, W1, b1, W2, b2):
    embeds = _sc_gather(emb.T, inputs).reshape(1, CONTEXT * EMBED_DIM)
    return _mlp_logits(embeds, W1, b1.reshape(1, HIDDEN),
                       W2, b2.reshape(1, VOCAB))


# speedup vs baseline: 1.0830x; 1.0830x over previous
import functools

import jax
import jax.numpy as jnp
from jax import lax
from jax.experimental import pallas as pl
from jax.experimental.pallas import tpu as pltpu
from jax.experimental.pallas import tpu_sc as plsc

VOCAB = 100000
EMBED_DIM = 64
CONTEXT = 200
HIDDEN = 128

_SC_NC = 2
_SC_NS = 16
_ROWS_PER_WORKER = 8
_ACTIVE_WORKERS = CONTEXT // _ROWS_PER_WORKER


def _sc_gather(embT, idx):
    mesh = plsc.VectorSubcoreMesh(core_axis_name="c", subcore_axis_name="s")

    @functools.partial(
        pl.kernel,
        mesh=mesh,
        out_type=jax.ShapeDtypeStruct((CONTEXT * EMBED_DIM,), jnp.float32),
        scratch_types=[
            pltpu.VMEM((16,), jnp.int32),
            pltpu.VMEM((_ROWS_PER_WORKER, EMBED_DIM, 128), jnp.float32),
            pltpu.VMEM((_ROWS_PER_WORKER * EMBED_DIM,), jnp.float32),
            pltpu.SemaphoreType.DMA,
        ],
        compiler_params=pltpu.CompilerParams(needs_layout_passes=False),
    )
    def k(embT_hbm, idx_hbm, out_hbm, idx_v, slab_v, rows_v, sem):
        wid = lax.axis_index("s") * _SC_NC + lax.axis_index("c")
        base = wid * _ROWS_PER_WORKER

        @pl.when(wid < _ACTIVE_WORKERS)
        def _():
            pltpu.sync_copy(idx_hbm.at[pl.ds(base, _ROWS_PER_WORKER)],
                            idx_v.at[pl.ds(0, _ROWS_PER_WORKER)])
            idx_vec = idx_v[...]
            copies = []
            for j in range(_ROWS_PER_WORKER):
                c0 = (idx_vec[j] // 128) * 128
                copies.append(pltpu.async_copy(
                    embT_hbm.at[:, pl.ds(c0, 128)], slab_v.at[j], sem))
            for c in copies:
                c.wait()
            for j in range(_ROWS_PER_WORKER):
                lane = idx_vec[j] % 128
                lane_vec = jnp.full((16,), lane, jnp.int32)
                j_vec = jnp.full((16,), j, jnp.int32)
                for ch in range(EMBED_DIM // 16):
                    d_vec = lax.iota(jnp.int32, 16) + 16 * ch
                    vals = plsc.load_gather(slab_v, [j_vec, d_vec, lane_vec])
                    rows_v[pl.ds(j * EMBED_DIM + 16 * ch, 16)] = vals
            pltpu.sync_copy(
                rows_v, out_hbm.at[pl.ds(base * EMBED_DIM,
                                         _ROWS_PER_WORKER * EMBED_DIM)])

    return k(embT, idx)


_BV = 4096
_NB = -(-VOCAB // _BV)


def _mlp_body(e_ref, W1_ref, b1_ref, W2_ref, b2_ref, out_ref,
              acc_ref, h_ref, m_ref, s_ref):
    i = pl.program_id(0)

    @pl.when(i == 0)
    def _():
        h = lax.dot_general(
            e_ref[...], W1_ref[...],
            dimension_numbers=(((1,), (1,)), ((), ())),
            preferred_element_type=jnp.float32,
        ) + b1_ref[...]
        h_ref[...] = jnp.maximum(h, 0.0).astype(jnp.bfloat16)
        m_ref[...] = jnp.full((1, 1), -jnp.inf, jnp.float32)
        s_ref[...] = jnp.zeros((1, 1), jnp.float32)

    logits = lax.dot_general(
        h_ref[...], W2_ref[...].astype(jnp.bfloat16),
        dimension_numbers=(((1,), (1,)), ((), ())),
        preferred_element_type=jnp.float32,
    ) + b2_ref[...]
    acc_ref[:, pl.ds(i * _BV, _BV)] = logits

    col = i * _BV + lax.broadcasted_iota(jnp.int32, (1, _BV), 1)
    masked = jnp.where(col < VOCAB, logits, -jnp.inf)

    m_old = m_ref[...]
    m_new = jnp.maximum(m_old, jnp.max(masked, axis=(0, 1), keepdims=True))
    s_ref[...] = (s_ref[...] * jnp.exp(m_old - m_new)
                  + jnp.sum(jnp.exp(masked - m_new), axis=(0, 1), keepdims=True))
    m_ref[...] = m_new

    @pl.when(i == _NB - 1)
    def _():
        lse = m_ref[...] + jnp.log(s_ref[...])
        out_ref[...] = acc_ref[:, :VOCAB] - lse


def _mlp_logits(embeds, W1, b1, W2, b2):
    return pl.pallas_call(
        _mlp_body,
        grid=(_NB,),
        in_specs=[
            pl.BlockSpec((1, CONTEXT * EMBED_DIM), lambda i: (0, 0)),
            pl.BlockSpec((HIDDEN, CONTEXT * EMBED_DIM), lambda i: (0, 0)),
            pl.BlockSpec((1, HIDDEN), lambda i: (0, 0)),
            pl.BlockSpec((_BV, HIDDEN), lambda i: (i, 0)),
            pl.BlockSpec((1, _BV), lambda i: (0, i)),
        ],
        out_specs=pl.BlockSpec((1, VOCAB), lambda i: (0, 0)),
        out_shape=jax.ShapeDtypeStruct((1, VOCAB), jnp.float32),
        scratch_shapes=[
            pltpu.VMEM((1, _BV * _NB), jnp.float32),
            pltpu.VMEM((1, HIDDEN), jnp.bfloat16),
            pltpu.VMEM((1, 1), jnp.float32),
            pltpu.VMEM((1, 1), jnp.float32),
        ],
    )(embeds, W1, b1, W2, b2)


def kernel(inputs, emb, W1, b1, W2, b2):
    embeds = _sc_gather(emb.T, inputs).reshape(1, CONTEXT * EMBED_DIM)
    return _mlp_logits(embeds, W1, b1.reshape(1, HIDDEN),
                       W2, b2.reshape(1, VOCAB))


# speedup vs baseline: 1.3272x; 1.2255x over previous
import functools

import jax
import jax.numpy as jnp
from jax import lax
from jax.experimental import pallas as pl
from jax.experimental.pallas import tpu as pltpu
from jax.experimental.pallas import tpu_sc as plsc

VOCAB = 100000
EMBED_DIM = 64
CONTEXT = 200
HIDDEN = 128

_SC_NC = 2
_SC_NS = 16
_ROWS_PER_WORKER = 8
_ACTIVE_WORKERS = CONTEXT // _ROWS_PER_WORKER


def _sc_gather(embT, idx):
    mesh = plsc.VectorSubcoreMesh(core_axis_name="c", subcore_axis_name="s")

    @functools.partial(
        pl.kernel,
        mesh=mesh,
        out_type=jax.ShapeDtypeStruct((CONTEXT * EMBED_DIM,), jnp.float32),
        scratch_types=[
            pltpu.VMEM((16,), jnp.int32),
            pltpu.VMEM((_ROWS_PER_WORKER, EMBED_DIM, 128), jnp.float32),
            pltpu.VMEM((_ROWS_PER_WORKER * EMBED_DIM,), jnp.float32),
            pltpu.SemaphoreType.DMA,
        ],
        compiler_params=pltpu.CompilerParams(needs_layout_passes=False),
    )
    def k(embT_hbm, idx_hbm, out_hbm, idx_v, slab_v, rows_v, sem):
        wid = lax.axis_index("s") * _SC_NC + lax.axis_index("c")
        base = wid * _ROWS_PER_WORKER

        @pl.when(wid < _ACTIVE_WORKERS)
        def _():
            pltpu.sync_copy(idx_hbm.at[pl.ds(base, _ROWS_PER_WORKER)],
                            idx_v.at[pl.ds(0, _ROWS_PER_WORKER)])
            idx_vec = idx_v[...]
            copies = []
            for j in range(_ROWS_PER_WORKER):
                c0 = (idx_vec[j] // 128) * 128
                copies.append(pltpu.async_copy(
                    embT_hbm.at[:, pl.ds(c0, 128)], slab_v.at[j], sem))
            for c in copies:
                c.wait()
            for j in range(_ROWS_PER_WORKER):
                lane = idx_vec[j] % 128
                lane_vec = jnp.full((16,), lane, jnp.int32)
                j_vec = jnp.full((16,), j, jnp.int32)
                for ch in range(EMBED_DIM // 16):
                    d_vec = lax.iota(jnp.int32, 16) + 16 * ch
                    vals = plsc.load_gather(slab_v, [j_vec, d_vec, lane_vec])
                    rows_v[pl.ds(j * EMBED_DIM + 16 * ch, 16)] = vals
            pltpu.sync_copy(
                rows_v, out_hbm.at[pl.ds(base * EMBED_DIM,
                                         _ROWS_PER_WORKER * EMBED_DIM)])

    return k(embT, idx)


_BV = 4096
_NB = -(-VOCAB // _BV)


_NBUF = 4


def _mlp_body(e_ref, W1_ref, b1_ref, W2_hbm, b2_ref, out_ref,
              bufs_ref, acc_ref, sems):
    def start(i):
        rows = min(_BV, VOCAB - i * _BV)
        return pltpu.make_async_copy(
            W2_hbm.at[pl.ds(i * _BV, rows), :],
            bufs_ref.at[i % _NBUF, pl.ds(0, rows), :],
            sems.at[i % _NBUF],
        ).start()

    for i in range(_NBUF - 1):
        start(i)

    h = lax.dot_general(
        e_ref[...], W1_ref[...],
        dimension_numbers=(((1,), (1,)), ((), ())),
        preferred_element_type=jnp.float32,
    ) + b1_ref[...]
    h = jnp.maximum(h, 0.0)

    m = jnp.full((1, 1), -jnp.inf, jnp.float32)
    s = jnp.zeros((1, 1), jnp.float32)
    for i in range(_NB):
        rows = min(_BV, VOCAB - i * _BV)
        pltpu.make_async_copy(
            W2_hbm.at[pl.ds(i * _BV, rows), :],
            bufs_ref.at[i % _NBUF, pl.ds(0, rows), :],
            sems.at[i % _NBUF],
        ).wait()
        logits = lax.dot_general(
            h, bufs_ref[i % _NBUF],
            dimension_numbers=(((1,), (1,)), ((), ())),
            preferred_element_type=jnp.float32,
        ) + b2_ref[:, pl.ds(i * _BV, _BV)]
        if i + _NBUF - 1 < _NB:
            start(i + _NBUF - 1)
        acc_ref[:, pl.ds(i * _BV, _BV)] = logits
        if rows < _BV:
            colv = lax.broadcasted_iota(jnp.int32, (1, _BV), 1)
            logits = jnp.where(colv < rows, logits, -jnp.inf)
        m_new = jnp.maximum(m, jnp.max(logits, axis=(0, 1), keepdims=True))
        s = (s * jnp.exp(m - m_new)
             + jnp.sum(jnp.exp(logits - m_new), axis=(0, 1), keepdims=True))
        m = m_new

    lse = m + jnp.log(s)
    out_ref[...] = acc_ref[:, :VOCAB] - lse


def _mlp_logits(embeds, W1, b1, W2, b2):
    return pl.pallas_call(
        _mlp_body,
        in_specs=[
            pl.BlockSpec((1, CONTEXT * EMBED_DIM), lambda: (0, 0)),
            pl.BlockSpec((HIDDEN, CONTEXT * EMBED_DIM), lambda: (0, 0)),
            pl.BlockSpec((1, HIDDEN), lambda: (0, 0)),
            pl.BlockSpec(memory_space=pl.ANY),
            pl.BlockSpec((1, _BV * _NB), lambda: (0, 0)),
        ],
        out_specs=pl.BlockSpec((1, VOCAB), lambda: (0, 0)),
        out_shape=jax.ShapeDtypeStruct((1, VOCAB), jnp.float32),
        scratch_shapes=[
            pltpu.VMEM((_NBUF, _BV, HIDDEN), jnp.float32),
            pltpu.VMEM((1, _BV * _NB), jnp.float32),
            pltpu.SemaphoreType.DMA((_NBUF,)),
        ],
    )(embeds, W1, b1, W2, b2)


def kernel(inputs, emb, W1, b1, W2, b2):
    embeds = _sc_gather(emb.T, inputs).reshape(1, CONTEXT * EMBED_DIM)
    b2p = jnp.pad(b2, (0, _BV * _NB - VOCAB)).reshape(1, _BV * _NB)
    return _mlp_logits(embeds, W1, b1.reshape(1, HIDDEN), W2, b2p)


# speedup vs baseline: 1.3779x; 1.0382x over previous
import functools

import jax
import jax.numpy as jnp
from jax import lax
from jax.experimental import pallas as pl
from jax.experimental.pallas import tpu as pltpu
from jax.experimental.pallas import tpu_sc as plsc

VOCAB = 100000
EMBED_DIM = 64
CONTEXT = 200
HIDDEN = 128

_SC_NC = 2
_SC_NS = 16
_ROWS_PER_WORKER = 8
_ACTIVE_WORKERS = CONTEXT // _ROWS_PER_WORKER


def _sc_gather(embT, idx):
    mesh = plsc.VectorSubcoreMesh(core_axis_name="c", subcore_axis_name="s")

    @functools.partial(
        pl.kernel,
        mesh=mesh,
        out_type=jax.ShapeDtypeStruct((1, CONTEXT * EMBED_DIM), jnp.float32),
        scratch_types=[
            pltpu.VMEM((16,), jnp.int32),
            pltpu.VMEM((_ROWS_PER_WORKER, EMBED_DIM, 128), jnp.float32),
            pltpu.VMEM((_ROWS_PER_WORKER * EMBED_DIM,), jnp.float32),
            pltpu.SemaphoreType.DMA,
        ],
        compiler_params=pltpu.CompilerParams(needs_layout_passes=False),
    )
    def k(embT_hbm, idx_hbm, out_hbm, idx_v, slab_v, rows_v, sem):
        wid = lax.axis_index("s") * _SC_NC + lax.axis_index("c")
        base = wid * _ROWS_PER_WORKER

        @pl.when(wid < _ACTIVE_WORKERS)
        def _():
            pltpu.sync_copy(idx_hbm.at[pl.ds(base, _ROWS_PER_WORKER)],
                            idx_v.at[pl.ds(0, _ROWS_PER_WORKER)])
            idx_vec = idx_v[...]
            copies = []
            for j in range(_ROWS_PER_WORKER):
                c0 = (idx_vec[j] // 128) * 128
                copies.append(pltpu.async_copy(
                    embT_hbm.at[:, pl.ds(c0, 128)], slab_v.at[j], sem))
            for c in copies:
                c.wait()
            for j in range(_ROWS_PER_WORKER):
                lane = idx_vec[j] % 128
                lane_vec = jnp.full((16,), lane, jnp.int32)
                j_vec = jnp.full((16,), j, jnp.int32)
                for ch in range(EMBED_DIM // 16):
                    d_vec = lax.iota(jnp.int32, 16) + 16 * ch
                    vals = plsc.load_gather(slab_v, [j_vec, d_vec, lane_vec])
                    rows_v[pl.ds(j * EMBED_DIM + 16 * ch, 16)] = vals
            pltpu.sync_copy(
                rows_v, out_hbm.at[0].at[pl.ds(base * EMBED_DIM,
                                               _ROWS_PER_WORKER * EMBED_DIM)])

    return k(embT, idx)


_BV = 4096
_NB = -(-VOCAB // _BV)


_NBUF = 6


def _mlp_body(e_ref, W1_ref, b1_ref, W2_hbm, b2_ref, out_ref,
              bufs_ref, acc_ref, sems):
    def start(i):
        rows = min(_BV, VOCAB - i * _BV)
        return pltpu.make_async_copy(
            W2_hbm.at[pl.ds(i * _BV, rows), :],
            bufs_ref.at[i % _NBUF, pl.ds(0, rows), :],
            sems.at[i % _NBUF],
        ).start()

    for i in range(_NBUF - 1):
        start(i)

    h = lax.dot_general(
        e_ref[...], W1_ref[...],
        dimension_numbers=(((1,), (1,)), ((), ())),
        preferred_element_type=jnp.float32,
    ) + b1_ref[...]
    h = jnp.maximum(h, 0.0)

    m = jnp.full((1, 1), -jnp.inf, jnp.float32)
    s = jnp.zeros((1, 1), jnp.float32)
    for i in range(_NB):
        rows = min(_BV, VOCAB - i * _BV)
        pltpu.make_async_copy(
            W2_hbm.at[pl.ds(i * _BV, rows), :],
            bufs_ref.at[i % _NBUF, pl.ds(0, rows), :],
            sems.at[i % _NBUF],
        ).wait()
        logits = lax.dot_general(
            h, bufs_ref[i % _NBUF],
            dimension_numbers=(((1,), (1,)), ((), ())),
            preferred_element_type=jnp.float32,
        ) + b2_ref[:, pl.ds(i * _BV, _BV)]
        if i + _NBUF - 1 < _NB:
            start(i + _NBUF - 1)
        acc_ref[:, pl.ds(i * _BV, _BV)] = logits
        if rows < _BV:
            colv = lax.broadcasted_iota(jnp.int32, (1, _BV), 1)
            logits = jnp.where(colv < rows, logits, -jnp.inf)
        m_new = jnp.maximum(m, jnp.max(logits, axis=(0, 1), keepdims=True))
        s = (s * jnp.exp(m - m_new)
             + jnp.sum(jnp.exp(logits - m_new), axis=(0, 1), keepdims=True))
        m = m_new

    lse = m + jnp.log(s)
    out_ref[...] = acc_ref[:, :VOCAB] - lse


def _mlp_logits(embeds, W1, b1, W2, b2):
    return pl.pallas_call(
        _mlp_body,
        in_specs=[
            pl.BlockSpec((1, CONTEXT * EMBED_DIM), lambda: (0, 0)),
            pl.BlockSpec((HIDDEN, CONTEXT * EMBED_DIM), lambda: (0, 0)),
            pl.BlockSpec((1, HIDDEN), lambda: (0, 0)),
            pl.BlockSpec(memory_space=pl.ANY),
            pl.BlockSpec((1, _BV * _NB), lambda: (0, 0)),
        ],
        out_specs=pl.BlockSpec((1, VOCAB), lambda: (0, 0)),
        out_shape=jax.ShapeDtypeStruct((1, VOCAB), jnp.float32),
        scratch_shapes=[
            pltpu.VMEM((_NBUF, _BV, HIDDEN), jnp.float32),
            pltpu.VMEM((1, _BV * _NB), jnp.float32),
            pltpu.SemaphoreType.DMA((_NBUF,)),
        ],
    )(embeds, W1, b1, W2, b2)


def kernel(inputs, emb, W1, b1, W2, b2):
    embeds = _sc_gather(emb.T, inputs)
    b2p = jnp.pad(b2, (0, _BV * _NB - VOCAB)).reshape(1, _BV * _NB)
    return _mlp_logits(embeds, W1, b1.reshape(1, HIDDEN), W2, b2p)
